# baseline (speedup 1.0000x reference)
import jax
import jax.numpy as jnp
from jax import lax
from jax.experimental import pallas as pl
from jax.experimental.pallas import tpu as pltpu

_B = 64
_N = 128
_F = 8
_SD = 14
_K = 4
_G = 8

_INTERPRET = False


def _graph_body(ve_ref, nodes_ref, edges_ref, mem_ref):
    ve = ve_ref[:]
    nodes = nodes_ref[:]
    diff = nodes - ve
    d2 = jnp.sum(diff * diff, axis=2, keepdims=True)

    io_n1 = lax.broadcasted_iota(jnp.int32, (_G, _N, 1), 1)
    m = jnp.min(d2, axis=1, keepdims=True)
    closest = jnp.min(jnp.where(d2 == m, io_n1, _N),
                      axis=1, keepdims=True).astype(jnp.int32)

    cost = edges_ref[:, 0]
    for c in range(1, 6):
        cost = jnp.minimum(cost, edges_ref[:, c])

    sub3 = lax.broadcasted_iota(jnp.int32, (_G, _N, _N), 1)
    lane3 = lax.broadcasted_iota(jnp.int32, (_G, _N, _N), 2)
    lane_row = lax.broadcasted_iota(jnp.int32, (_G, 1, _N), 2)
    eye = sub3 == lane3
    inf = jnp.float32(jnp.inf)

    d_row = jnp.min(jnp.where(sub3 == closest, cost, inf),
                    axis=1, keepdims=True)
    d_row = jnp.where(lane_row == closest, jnp.float32(0.0), d_row)

    def bf_cond(carry):
        _, changed, it = carry
        return jnp.logical_and(changed, it < _N - 1)

    def bf_body(carry):
        d, _, it = carry
        d_col = jnp.min(jnp.where(eye, jnp.broadcast_to(d, (_G, _N, _N)), inf),
                        axis=2, keepdims=True)
        relaxed = jnp.min(d_col + cost, axis=1, keepdims=True)
        new_d = jnp.minimum(d, relaxed)
        return new_d, jnp.any(new_d < d), it + jnp.int32(1)

    d_row, _, _ = lax.while_loop(
        bf_cond, bf_body, (d_row, jnp.array(True), jnp.int32(0)))

    arows = []
    for c in range(6):
        ec = edges_ref[:, c]
        arows.append(jnp.sum(jnp.where(sub3 == closest, ec, 0.0),
                             axis=1, keepdims=True))

    nodes14 = jnp.concatenate(
        [nodes, jnp.zeros((_G, _N, _SD - _F), jnp.float32)], axis=2)
    node_rowio = lax.broadcasted_iota(jnp.int32, (_G, _N, _SD), 1)
    lane14 = lax.broadcasted_iota(jnp.int32, (_G, 1, _SD), 2)

    dw = d_row
    rows = []
    for k in range(_K):
        mk = jnp.min(dw, axis=2, keepdims=True)
        ik = jnp.min(jnp.where(dw == mk, lane_row, _N),
                     axis=2, keepdims=True).astype(jnp.int32)
        dw = jnp.where(lane_row == ik, inf, dw)
        row = jnp.sum(jnp.where(node_rowio == ik, nodes14, 0.0),
                      axis=1, keepdims=True)
        for c in range(6):
            val = jnp.sum(jnp.where(lane_row == ik, arows[c], 0.0),
                          axis=2, keepdims=True)
            row = row + jnp.where(lane14 == _F + c, val, 0.0)
        rows.append(row)
    mem_ref[:] = jnp.concatenate(rows, axis=1)


def _graph_call(vision_enc, nodes, edges_t):
    return pl.pallas_call(
        _graph_body,
        grid=(_B // _G,),
        in_specs=[
            pl.BlockSpec((_G, 1, _F), lambda b: (b, 0, 0)),
            pl.BlockSpec((_G, _N, _F), lambda b: (b, 0, 0)),
            pl.BlockSpec((_G, 6, _N, _N), lambda b: (b, 0, 0, 0)),
        ],
        out_specs=pl.BlockSpec((_G, _K, _SD), lambda b: (b, 0, 0)),
        out_shape=jax.ShapeDtypeStruct((_B, _K, _SD), jnp.float32),
        compiler_params=pltpu.CompilerParams(
            dimension_semantics=("arbitrary",)),
        interpret=_INTERPRET,
    )(vision_enc, nodes, edges_t)


def _tail_body(tcol_ref, mem_ref,
               t1w_ref, t1b_ref, t2w_ref, t2b_ref,
               wq_ref, bq_ref, wk_ref, bk_ref, wv_ref, bv_ref,
               wo_ref, bo_ref, ln1g_ref, ln1b_ref,
               f1w_ref, f1b_ref, f2w_ref, f2b_ref,
               ln2g_ref, ln2b_ref,
               h1w_ref, h1b_ref, h2w_ref, h2b_ref, h3w_ref, h3b_ref,
               out_ref):
    tcol = tcol_ref[:]
    t = jnp.maximum(tcol @ t1w_ref[:] + t1b_ref[:], 0.0)
    te = t @ t2w_ref[:] + t2b_ref[:]
    mem = mem_ref[:]
    s = jnp.concatenate([te, mem], axis=0)

    q = s @ wq_ref[:] + bq_ref[:]
    k = s @ wk_ref[:] + bk_ref[:]
    v = s @ wv_ref[:] + bv_ref[:]
    scores = lax.dot_general(q, k, (((1,), (1,)), ((), ())))
    scores = scores / jnp.sqrt(jnp.float32(_SD))

    rio = lax.broadcasted_iota(jnp.int32, (5 * _B, 1), 0)
    cio = lax.broadcasted_iota(jnp.int32, (1, 5 * _B), 1)
    g_r = jnp.where(rio < _B, rio, (rio - _B) // 4)
    g_c = jnp.where(cio < _B, cio, (cio - _B) // 4)
    mask = g_r == g_c
    neg = jnp.float32(-jnp.inf)
    scores = jnp.where(mask, scores, neg)
    mx = jnp.max(scores, axis=1, keepdims=True)
    e = jnp.exp(scores - mx)
    attn_w = e / jnp.sum(e, axis=1, keepdims=True)
    att = attn_w @ v
    att = att @ wo_ref[:] + bo_ref[:]

    def ln(x, g, b):
        mu = jnp.mean(x, axis=1, keepdims=True)
        var = jnp.mean((x - mu) ** 2, axis=1, keepdims=True)
        return (x - mu) / jnp.sqrt(var + 1e-5) * g + b

    s1 = ln(s + att, ln1g_ref[:], ln1b_ref[:])
    ff = jnp.maximum(s1 @ f1w_ref[:] + f1b_ref[:], 0.0)
    ff = ff @ f2w_ref[:] + f2b_ref[:]
    s2 = ln(s1 + ff, ln2g_ref[:], ln2b_ref[:])

    t_final = s2[0:_B, :]
    m_final = s2[_B:, :]
    prow = lax.broadcasted_iota(jnp.int32, (_B, 4 * _B), 0)
    pcol = lax.broadcasted_iota(jnp.int32, (_B, 4 * _B), 1)
    pmat = jnp.where(prow == pcol // 4, jnp.float32(0.25), jnp.float32(0.0))
    m_mean = pmat @ m_final
    pooled = jnp.concatenate([t_final, m_mean], axis=1)

    h = jnp.maximum(pooled @ h1w_ref[:] + h1b_ref[:], 0.0)
    h = jnp.maximum(h @ h2w_ref[:] + h2b_ref[:], 0.0)
    out_ref[:] = h @ h3w_ref[:] + h3b_ref[:]


def _tail_call(tcol, mem2d, p):
    def t2(name):
        return p[name].T
    def b2(name):
        return p[name][None, :]
    operands = [
        tcol, mem2d,
        t2('tenc1_w'), b2('tenc1_b'), t2('tenc2_w'), b2('tenc2_b'),
        t2('wq'), b2('bq'), t2('wk'), b2('bk'), t2('wv'), b2('bv'),
        t2('wo'), b2('bo'), b2('ln1_g'), b2('ln1_b'),
        t2('ff1_w'), b2('ff1_b'), t2('ff2_w'), b2('ff2_b'),
        b2('ln2_g'), b2('ln2_b'),
        t2('h1_w'), b2('h1_b'), t2('h2_w'), b2('h2_b'),
        t2('h3_w'), b2('h3_b'),
    ]
    return pl.pallas_call(
        _tail_body,
        out_shape=jax.ShapeDtypeStruct((_B, 6), jnp.float32),
        interpret=_INTERPRET,
    )(*operands)


_GC = 32


def _conv_body(x_ref, bw1a, bw1b, bw1c, b1_ref, bw2a, bw2b, bw2c, b2_ref,
               bw3a, bw3b, bw3c, b3_ref, bp1_ref, bp2_ref,
               wbig_ref, mdiag_ref, sel_ref, vb_ref, out_ref):
    g = _GC
    bw1 = (bw1a, bw1b, bw1c)
    bw2 = (bw2a, bw2b, bw2c)
    bw3 = (bw3a, bw3b, bw3c)
    xp = x_ref[:] - 0.5

    o1 = jnp.zeros((g * 62, 992), jnp.float32)
    for dy in range(3):
        s = xp[:, dy:dy + 62, :].reshape(g * 62, 192)
        o1 = o1 + s @ bw1[dy][:]
    o1 = o1 + b1_ref[:]
    o1 = o1.reshape(g * 31, 2, 992)
    o1 = o1[:, 0, :] + o1[:, 1, :]
    p1 = jnp.maximum(o1 @ bp1_ref[:], 0.0)

    zlane = jnp.zeros((g * 31, 16), jnp.float32)
    p1 = jnp.concatenate([zlane, p1, zlane], axis=1)
    p1 = p1.reshape(g, 31, 528)
    zrow = jnp.zeros((g, 1, 528), jnp.float32)
    p1 = jnp.concatenate([zrow, p1, zrow], axis=1)
    o2 = jnp.zeros((g * 31, 992), jnp.float32)
    for dy in range(3):
        s = p1[:, dy:dy + 31, :].reshape(g * 31, 528)
        o2 = o2 + s @ bw2[dy][:]
    o2 = o2 + b2_ref[:]
    o2 = o2.reshape(g, 31, 992)[:, 0:30, :].reshape(g * 15, 2, 992)
    o2 = o2[:, 0, :] + o2[:, 1, :]
    p2 = jnp.maximum(o2 @ bp2_ref[:], 0.0)

    zlane3 = jnp.zeros((g * 15, 32), jnp.float32)
    p2 = jnp.concatenate([zlane3, p2, zlane3], axis=1)
    p2 = p2.reshape(g, 15, 544)
    zrow3 = jnp.zeros((g, 1, 544), jnp.float32)
    p2 = jnp.concatenate([zrow3, p2, zrow3], axis=1)
    o3 = jnp.zeros((g * 15, 480), jnp.float32)
    for dy in range(3):
        s = p2[:, dy:dy + 15, :].reshape(g * 15, 544)
        o3 = o3 + s @ bw3[dy][:]
    h3 = jnp.maximum(o3 + b3_ref[:], 0.0)

    t = h3 @ wbig_ref[:]
    t = t.reshape(g, 15, 120) * mdiag_ref[:]
    v = jnp.sum(t, axis=1)
    ve = v @ sel_ref[:] + vb_ref[:]
    out_ref[:] = ve.reshape(g, 1, _F)


def _conv_call(x_hwc, p):
    w1, w2, w3 = p['conv1_w'], p['conv2_w'], p['conv3_w']

    def bands(w, n_in, n_out, c_in, c_out):
        wi = jnp.arange(n_in)[:, None]
        wo = jnp.arange(n_out)[None, :]
        d = jnp.stack([(wi == wo + dx).astype(jnp.float32)
                       for dx in range(3)])
        m = jnp.einsum('xpw,fcyx->ypcwf', d, w,
                       precision=lax.Precision.HIGHEST)
        m = m.reshape(3, n_in * c_in, n_out * c_out)
        return m[0], m[1], m[2]

    bw1 = bands(w1, 64, 62, 3, 16)
    bw2 = bands(w2, 33, 31, 16, 32)
    bw3 = bands(w3, 17, 15, 32, 32)

    def poolmat(n_in, n_keep, c):
        wi = jnp.arange(n_in)[:, None, None, None]
        wo = jnp.arange(n_keep)[None, None, :, None]
        ci = jnp.arange(c)[None, :, None, None]
        co = jnp.arange(c)[None, None, None, :]
        m = ((wi // 2 == wo) & (wi < 2 * n_keep) & (ci == co))
        return m.astype(jnp.float32).reshape(n_in * c, n_keep * c) * 0.25

    bp1 = poolmat(62, 31, 16)
    bp2 = poolmat(31, 15, 32)

    b1 = jnp.tile(p['conv1_b'], 62)[None, :]
    b2 = jnp.tile(p['conv2_b'], 31)[None, :]
    b3 = jnp.tile(p['conv3_b'], 15)[None, :]

    wbig = p['venc_w'].reshape(_F, 32, 15, 15).transpose(3, 1, 0, 2)
    wbig = wbig.reshape(480, 120)
    bb = jnp.arange(120)
    hh = jnp.arange(15)[:, None]
    mdiag = (bb[None, :] % 15 == hh).astype(jnp.float32)
    sel = (bb[:, None] // 15 == jnp.arange(_F)[None, :]).astype(jnp.float32)
    vb = p['venc_b'][None, :]

    ops = [x_hwc, *bw1, b1, *bw2, b2, *bw3, b3, bp1, bp2,
           wbig, mdiag, sel, vb]
    specs = [pl.BlockSpec((_GC, 64, 192), lambda b: (b, 0, 0))]
    for o in ops[1:]:
        specs.append(pl.BlockSpec(o.shape, lambda b: (0,) * o.ndim))
    return pl.pallas_call(
        _conv_body,
        grid=(_B // _GC,),
        in_specs=specs,
        out_specs=pl.BlockSpec((_GC, 1, _F), lambda b: (b, 0, 0)),
        out_shape=jax.ShapeDtypeStruct((_B, 1, _F), jnp.float32),
        interpret=_INTERPRET,
    )(*ops)


def kernel(x, nodes, edges, params):
    p = params
    tcol = x[:, :, 0, 0]
    x_hwc = x.transpose(0, 2, 3, 1).reshape(_B, 64, 192)
    vision_enc = _conv_call(x_hwc, p)

    edges_t = jnp.moveaxis(edges, -1, 1)
    mem_seq = _graph_call(vision_enc, nodes, edges_t)
    mem2d = mem_seq.reshape(_B * _K, _SD)
    return _tail_call(tcol, mem2d, params)


# speedup vs baseline: 2.9529x; 2.9529x over previous
import jax
import jax.numpy as jnp
from jax import lax
from jax.experimental import pallas as pl
from jax.experimental.pallas import tpu as pltpu

_B = 64
_N = 128
_F = 8
_SD = 14
_K = 4
_G = 16

_INTERPRET = False


def _graph_body(ve_ref, nodes_ref, edges_ref, mem_ref):
    ve = ve_ref[:]
    nodes = nodes_ref[:]
    diff = nodes - ve
    d2 = jnp.sum(diff * diff, axis=2, keepdims=True)

    io_n1 = lax.broadcasted_iota(jnp.int32, (_G, _N, 1), 1)
    m = jnp.min(d2, axis=1, keepdims=True)
    closest = jnp.min(jnp.where(d2 == m, io_n1, _N),
                      axis=1, keepdims=True).astype(jnp.int32)

    cost = edges_ref[:, 0]
    for c in range(1, 6):
        cost = jnp.minimum(cost, edges_ref[:, c])

    sub3 = lax.broadcasted_iota(jnp.int32, (_G, _N, _N), 1)
    lane3 = lax.broadcasted_iota(jnp.int32, (_G, _N, _N), 2)
    lane_row = lax.broadcasted_iota(jnp.int32, (_G, 1, _N), 2)
    eye = sub3 == lane3
    inf = jnp.float32(jnp.inf)

    d_row = jnp.min(jnp.where(sub3 == closest, cost, inf),
                    axis=1, keepdims=True)
    d_row = jnp.where(lane_row == closest, jnp.float32(0.0), d_row)

    def bf_cond(carry):
        _, changed, it = carry
        return jnp.logical_and(changed, it < _N - 1)

    def bf_body(carry):
        d, _, it = carry
        d_col = jnp.min(jnp.where(eye, jnp.broadcast_to(d, (_G, _N, _N)), inf),
                        axis=2, keepdims=True)
        relaxed = jnp.min(d_col + cost, axis=1, keepdims=True)
        new_d = jnp.minimum(d, relaxed)
        return new_d, jnp.any(new_d < d), it + jnp.int32(1)

    d_row, _, _ = lax.while_loop(
        bf_cond, bf_body, (d_row, jnp.array(True), jnp.int32(0)))

    arows = []
    for c in range(6):
        ec = edges_ref[:, c]
        arows.append(jnp.sum(jnp.where(sub3 == closest, ec, 0.0),
                             axis=1, keepdims=True))

    nodes14 = jnp.concatenate(
        [nodes, jnp.zeros((_G, _N, _SD - _F), jnp.float32)], axis=2)
    node_rowio = lax.broadcasted_iota(jnp.int32, (_G, _N, _SD), 1)
    lane14 = lax.broadcasted_iota(jnp.int32, (_G, 1, _SD), 2)

    dw = d_row
    rows = []
    for k in range(_K):
        mk = jnp.min(dw, axis=2, keepdims=True)
        ik = jnp.min(jnp.where(dw == mk, lane_row, _N),
                     axis=2, keepdims=True).astype(jnp.int32)
        dw = jnp.where(lane_row == ik, inf, dw)
        row = jnp.sum(jnp.where(node_rowio == ik, nodes14, 0.0),
                      axis=1, keepdims=True)
        for c in range(6):
            val = jnp.sum(jnp.where(lane_row == ik, arows[c], 0.0),
                          axis=2, keepdims=True)
            row = row + jnp.where(lane14 == _F + c, val, 0.0)
        rows.append(row)
    mem_ref[:] = jnp.concatenate(rows, axis=1)


def _graph_call(vision_enc, nodes, edges_t):
    return pl.pallas_call(
        _graph_body,
        grid=(_B // _G,),
        in_specs=[
            pl.BlockSpec((_G, 1, _F), lambda b: (b, 0, 0)),
            pl.BlockSpec((_G, _N, _F), lambda b: (b, 0, 0)),
            pl.BlockSpec((_G, 6, _N, _N), lambda b: (b, 0, 0, 0)),
        ],
        out_specs=pl.BlockSpec((_G, _K, _SD), lambda b: (b, 0, 0)),
        out_shape=jax.ShapeDtypeStruct((_B, _K, _SD), jnp.float32),
        compiler_params=pltpu.CompilerParams(
            dimension_semantics=("arbitrary",)),
        interpret=_INTERPRET,
    )(vision_enc, nodes, edges_t)


def _tail_body(tcol_ref, mem_ref,
               t1w_ref, t1b_ref, t2w_ref, t2b_ref,
               wq_ref, bq_ref, wk_ref, bk_ref, wv_ref, bv_ref,
               wo_ref, bo_ref, ln1g_ref, ln1b_ref,
               f1w_ref, f1b_ref, f2w_ref, f2b_ref,
               ln2g_ref, ln2b_ref,
               h1w_ref, h1b_ref, h2w_ref, h2b_ref, h3w_ref, h3b_ref,
               out_ref):
    tcol = tcol_ref[:]
    t = jnp.maximum(tcol @ t1w_ref[:] + t1b_ref[:], 0.0)
    te = t @ t2w_ref[:] + t2b_ref[:]
    mem = mem_ref[:]
    s = jnp.concatenate([te, mem], axis=0)

    q = s @ wq_ref[:] + bq_ref[:]
    k = s @ wk_ref[:] + bk_ref[:]
    v = s @ wv_ref[:] + bv_ref[:]
    scores = lax.dot_general(q, k, (((1,), (1,)), ((), ())))
    scores = scores / jnp.sqrt(jnp.float32(_SD))

    rio = lax.broadcasted_iota(jnp.int32, (5 * _B, 1), 0)
    cio = lax.broadcasted_iota(jnp.int32, (1, 5 * _B), 1)
    g_r = jnp.where(rio < _B, rio, (rio - _B) // 4)
    g_c = jnp.where(cio < _B, cio, (cio - _B) // 4)
    mask = g_r == g_c
    neg = jnp.float32(-jnp.inf)
    scores = jnp.where(mask, scores, neg)
    mx = jnp.max(scores, axis=1, keepdims=True)
    e = jnp.exp(scores - mx)
    attn_w = e / jnp.sum(e, axis=1, keepdims=True)
    att = attn_w @ v
    att = att @ wo_ref[:] + bo_ref[:]

    def ln(x, g, b):
        mu = jnp.mean(x, axis=1, keepdims=True)
        var = jnp.mean((x - mu) ** 2, axis=1, keepdims=True)
        return (x - mu) / jnp.sqrt(var + 1e-5) * g + b

    s1 = ln(s + att, ln1g_ref[:], ln1b_ref[:])
    ff = jnp.maximum(s1 @ f1w_ref[:] + f1b_ref[:], 0.0)
    ff = ff @ f2w_ref[:] + f2b_ref[:]
    s2 = ln(s1 + ff, ln2g_ref[:], ln2b_ref[:])

    t_final = s2[0:_B, :]
    m_final = s2[_B:, :]
    prow = lax.broadcasted_iota(jnp.int32, (_B, 4 * _B), 0)
    pcol = lax.broadcasted_iota(jnp.int32, (_B, 4 * _B), 1)
    pmat = jnp.where(prow == pcol // 4, jnp.float32(0.25), jnp.float32(0.0))
    m_mean = pmat @ m_final
    pooled = jnp.concatenate([t_final, m_mean], axis=1)

    h = jnp.maximum(pooled @ h1w_ref[:] + h1b_ref[:], 0.0)
    h = jnp.maximum(h @ h2w_ref[:] + h2b_ref[:], 0.0)
    out_ref[:] = h @ h3w_ref[:] + h3b_ref[:]


def _tail_call(tcol, mem2d, p):
    def t2(name):
        return p[name].T
    def b2(name):
        return p[name][None, :]
    operands = [
        tcol, mem2d,
        t2('tenc1_w'), b2('tenc1_b'), t2('tenc2_w'), b2('tenc2_b'),
        t2('wq'), b2('bq'), t2('wk'), b2('bk'), t2('wv'), b2('bv'),
        t2('wo'), b2('bo'), b2('ln1_g'), b2('ln1_b'),
        t2('ff1_w'), b2('ff1_b'), t2('ff2_w'), b2('ff2_b'),
        b2('ln2_g'), b2('ln2_b'),
        t2('h1_w'), b2('h1_b'), t2('h2_w'), b2('h2_b'),
        t2('h3_w'), b2('h3_b'),
    ]
    return pl.pallas_call(
        _tail_body,
        out_shape=jax.ShapeDtypeStruct((_B, 6), jnp.float32),
        interpret=_INTERPRET,
    )(*operands)


def _conv2d(x, w, b, padding):
    out = lax.conv_general_dilated(x, w, window_strides=(1, 1), padding=padding,
                                   dimension_numbers=('NCHW', 'OIHW', 'NCHW'))
    return out + b[None, :, None, None]


def _avgpool2(x):
    s = lax.reduce_window(x, 0.0, lax.add, (1, 1, 2, 2), (1, 1, 2, 2), 'VALID')
    return s / 4.0


def kernel(x, nodes, edges, params):
    p = params
    tcol = x[:, :, 0, 0]
    xv = x - 0.5
    h = jax.nn.relu(_avgpool2(_conv2d(xv, p['conv1_w'], p['conv1_b'], 'VALID')))
    h = jax.nn.relu(_avgpool2(_conv2d(h, p['conv2_w'], p['conv2_b'], 'SAME')))
    h = jax.nn.relu(_conv2d(h, p['conv3_w'], p['conv3_b'], 'SAME'))
    h = h.reshape(_B, -1)
    vision_enc = h @ p['venc_w'].T + p['venc_b']

    edges_t = jnp.moveaxis(edges, -1, 1)
    mem_seq = _graph_call(vision_enc[:, None, :], nodes, edges_t)
    mem2d = mem_seq.reshape(_B * _K, _SD)
    return _tail_call(tcol, mem2d, params)


# speedup vs baseline: 3.0356x; 1.0280x over previous
import jax
import jax.numpy as jnp
from jax import lax
from jax.experimental import pallas as pl
from jax.experimental.pallas import tpu as pltpu

_B = 64
_N = 128
_F = 8
_SD = 14
_K = 4
_G = 32

_INTERPRET = False


def _graph_body(ve_ref, nodes_ref, edges_ref, mem_ref):
    ve = ve_ref[:]
    nodes = nodes_ref[:]
    diff = nodes - ve
    d2 = jnp.sum(diff * diff, axis=2, keepdims=True)

    io_n1 = lax.broadcasted_iota(jnp.int32, (_G, _N, 1), 1)
    m = jnp.min(d2, axis=1, keepdims=True)
    closest = jnp.min(jnp.where(d2 == m, io_n1, _N),
                      axis=1, keepdims=True).astype(jnp.int32)

    cost = edges_ref[:, 0]
    for c in range(1, 6):
        cost = jnp.minimum(cost, edges_ref[:, c])

    sub3 = lax.broadcasted_iota(jnp.int32, (_G, _N, _N), 1)
    lane3 = lax.broadcasted_iota(jnp.int32, (_G, _N, _N), 2)
    lane_row = lax.broadcasted_iota(jnp.int32, (_G, 1, _N), 2)
    eye = sub3 == lane3
    inf = jnp.float32(jnp.inf)

    d_row = jnp.min(jnp.where(sub3 == closest, cost, inf),
                    axis=1, keepdims=True)
    d_row = jnp.where(lane_row == closest, jnp.float32(0.0), d_row)

    def bf_cond(carry):
        _, changed, it = carry
        return jnp.logical_and(changed, it < _N - 1)

    def bf_body(carry):
        d, _, it = carry
        d_col = jnp.min(jnp.where(eye, jnp.broadcast_to(d, (_G, _N, _N)), inf),
                        axis=2, keepdims=True)
        relaxed = jnp.min(d_col + cost, axis=1, keepdims=True)
        new_d = jnp.minimum(d, relaxed)
        return new_d, jnp.any(new_d < d), it + jnp.int32(1)

    d_row, _, _ = lax.while_loop(
        bf_cond, bf_body, (d_row, jnp.array(True), jnp.int32(0)))

    arows = []
    for c in range(6):
        ec = edges_ref[:, c]
        arows.append(jnp.sum(jnp.where(sub3 == closest, ec, 0.0),
                             axis=1, keepdims=True))

    nodes14 = jnp.concatenate(
        [nodes, jnp.zeros((_G, _N, _SD - _F), jnp.float32)], axis=2)
    node_rowio = lax.broadcasted_iota(jnp.int32, (_G, _N, _SD), 1)
    lane14 = lax.broadcasted_iota(jnp.int32, (_G, 1, _SD), 2)

    dw = d_row
    rows = []
    for k in range(_K):
        mk = jnp.min(dw, axis=2, keepdims=True)
        ik = jnp.min(jnp.where(dw == mk, lane_row, _N),
                     axis=2, keepdims=True).astype(jnp.int32)
        dw = jnp.where(lane_row == ik, inf, dw)
        row = jnp.sum(jnp.where(node_rowio == ik, nodes14, 0.0),
                      axis=1, keepdims=True)
        for c in range(6):
            val = jnp.sum(jnp.where(lane_row == ik, arows[c], 0.0),
                          axis=2, keepdims=True)
            row = row + jnp.where(lane14 == _F + c, val, 0.0)
        rows.append(row)
    mem_ref[:] = jnp.concatenate(rows, axis=1)


def _graph_call(vision_enc, nodes, edges_t):
    return pl.pallas_call(
        _graph_body,
        grid=(_B // _G,),
        in_specs=[
            pl.BlockSpec((_G, 1, _F), lambda b: (b, 0, 0)),
            pl.BlockSpec((_G, _N, _F), lambda b: (b, 0, 0)),
            pl.BlockSpec((_G, 6, _N, _N), lambda b: (b, 0, 0, 0)),
        ],
        out_specs=pl.BlockSpec((_G, _K, _SD), lambda b: (b, 0, 0)),
        out_shape=jax.ShapeDtypeStruct((_B, _K, _SD), jnp.float32),
        compiler_params=pltpu.CompilerParams(
            dimension_semantics=("arbitrary",)),
        interpret=_INTERPRET,
    )(vision_enc, nodes, edges_t)


def _tail_body(tcol_ref, mem_ref,
               t1w_ref, t1b_ref, t2w_ref, t2b_ref,
               wq_ref, bq_ref, wk_ref, bk_ref, wv_ref, bv_ref,
               wo_ref, bo_ref, ln1g_ref, ln1b_ref,
               f1w_ref, f1b_ref, f2w_ref, f2b_ref,
               ln2g_ref, ln2b_ref,
               h1w_ref, h1b_ref, h2w_ref, h2b_ref, h3w_ref, h3b_ref,
               out_ref):
    tcol = tcol_ref[:]
    t = jnp.maximum(tcol @ t1w_ref[:] + t1b_ref[:], 0.0)
    te = t @ t2w_ref[:] + t2b_ref[:]
    mem = mem_ref[:]
    s = jnp.concatenate([te, mem], axis=0)

    q = s @ wq_ref[:] + bq_ref[:]
    k = s @ wk_ref[:] + bk_ref[:]
    v = s @ wv_ref[:] + bv_ref[:]
    scores = lax.dot_general(q, k, (((1,), (1,)), ((), ())))
    scores = scores / jnp.sqrt(jnp.float32(_SD))

    rio = lax.broadcasted_iota(jnp.int32, (5 * _B, 1), 0)
    cio = lax.broadcasted_iota(jnp.int32, (1, 5 * _B), 1)
    g_r = jnp.where(rio < _B, rio, (rio - _B) // 4)
    g_c = jnp.where(cio < _B, cio, (cio - _B) // 4)
    mask = g_r == g_c
    neg = jnp.float32(-jnp.inf)
    scores = jnp.where(mask, scores, neg)
    mx = jnp.max(scores, axis=1, keepdims=True)
    e = jnp.exp(scores - mx)
    attn_w = e / jnp.sum(e, axis=1, keepdims=True)
    att = attn_w @ v
    att = att @ wo_ref[:] + bo_ref[:]

    def ln(x, g, b):
        mu = jnp.mean(x, axis=1, keepdims=True)
        var = jnp.mean((x - mu) ** 2, axis=1, keepdims=True)
        return (x - mu) / jnp.sqrt(var + 1e-5) * g + b

    s1 = ln(s + att, ln1g_ref[:], ln1b_ref[:])
    ff = jnp.maximum(s1 @ f1w_ref[:] + f1b_ref[:], 0.0)
    ff = ff @ f2w_ref[:] + f2b_ref[:]
    s2 = ln(s1 + ff, ln2g_ref[:], ln2b_ref[:])

    t_final = s2[0:_B, :]
    m_final = s2[_B:, :]
    prow = lax.broadcasted_iota(jnp.int32, (_B, 4 * _B), 0)
    pcol = lax.broadcasted_iota(jnp.int32, (_B, 4 * _B), 1)
    pmat = jnp.where(prow == pcol // 4, jnp.float32(0.25), jnp.float32(0.0))
    m_mean = pmat @ m_final
    pooled = jnp.concatenate([t_final, m_mean], axis=1)

    h = jnp.maximum(pooled @ h1w_ref[:] + h1b_ref[:], 0.0)
    h = jnp.maximum(h @ h2w_ref[:] + h2b_ref[:], 0.0)
    out_ref[:] = h @ h3w_ref[:] + h3b_ref[:]


def _tail_call(tcol, mem2d, p):
    def t2(name):
        return p[name].T
    def b2(name):
        return p[name][None, :]
    operands = [
        tcol, mem2d,
        t2('tenc1_w'), b2('tenc1_b'), t2('tenc2_w'), b2('tenc2_b'),
        t2('wq'), b2('bq'), t2('wk'), b2('bk'), t2('wv'), b2('bv'),
        t2('wo'), b2('bo'), b2('ln1_g'), b2('ln1_b'),
        t2('ff1_w'), b2('ff1_b'), t2('ff2_w'), b2('ff2_b'),
        b2('ln2_g'), b2('ln2_b'),
        t2('h1_w'), b2('h1_b'), t2('h2_w'), b2('h2_b'),
        t2('h3_w'), b2('h3_b'),
    ]
    return pl.pallas_call(
        _tail_body,
        out_shape=jax.ShapeDtypeStruct((_B, 6), jnp.float32),
        interpret=_INTERPRET,
    )(*operands)


def _conv2d(x, w, b, padding):
    out = lax.conv_general_dilated(x, w, window_strides=(1, 1), padding=padding,
                                   dimension_numbers=('NCHW', 'OIHW', 'NCHW'))
    return out + b[None, :, None, None]


def _avgpool2(x):
    s = lax.reduce_window(x, 0.0, lax.add, (1, 1, 2, 2), (1, 1, 2, 2), 'VALID')
    return s / 4.0


def kernel(x, nodes, edges, params):
    p = params
    tcol = x[:, :, 0, 0]
    xv = x - 0.5
    h = jax.nn.relu(_avgpool2(_conv2d(xv, p['conv1_w'], p['conv1_b'], 'VALID')))
    h = jax.nn.relu(_avgpool2(_conv2d(h, p['conv2_w'], p['conv2_b'], 'SAME')))
    h = jax.nn.relu(_conv2d(h, p['conv3_w'], p['conv3_b'], 'SAME'))
    h = h.reshape(_B, -1)
    vision_enc = h @ p['venc_w'].T + p['venc_b']

    edges_t = jnp.moveaxis(edges, -1, 1)
    mem_seq = _graph_call(vision_enc[:, None, :], nodes, edges_t)
    mem2d = mem_seq.reshape(_B * _K, _SD)
    return _tail_call(tcol, mem2d, params)
